# baseline (speedup 1.0000x reference)
import functools

import jax
import jax.numpy as jnp
from jax import lax
from jax.experimental import pallas as pl
from jax.experimental.pallas import tpu as pltpu
from jax.experimental.pallas import tpu_sc as plsc

BATCH = 16384
DIM = 32
LANES = 16
TILE = 128
DEPTH = 8

_info = plsc.get_sparse_core_info()
_NC = _info.num_cores
_NS = _info.num_subcores
_NW = _NC * _NS
_BPW = BATCH // _NW
_NG = _BPW // LANES

_mesh = plsc.VectorSubcoreMesh(core_axis_name="c", subcore_axis_name="s")


@functools.partial(
    pl.kernel,
    mesh=_mesh,
    compiler_params=pltpu.CompilerParams(needs_layout_passes=False),
    out_type=jax.ShapeDtypeStruct((DIM, BATCH), jnp.float32),
    scratch_types=[
        pltpu.VMEM((_BPW,), jnp.int32),
        pltpu.VMEM((_BPW,), jnp.int32),
        pltpu.VMEM((DEPTH, DIM, TILE), jnp.float32),
        pltpu.VMEM((DEPTH, DIM, TILE), jnp.float32),
        pltpu.VMEM((DIM, _BPW), jnp.float32),
        [pltpu.SemaphoreType.DMA] * DEPTH,
        [pltpu.SemaphoreType.DMA] * DEPTH,
    ],
)
def _mf_sc(uidx_hbm, iidx_hbm, ut_hbm, it_hbm, out_hbm,
           uidx_v, iidx_v, ubufs, ibufs, ocols, semus, semis):
    wid = lax.axis_index("s") * _NC + lax.axis_index("c")
    base = wid * _BPW

    pltpu.sync_copy(uidx_hbm.at[pl.ds(base, _BPW)], uidx_v)
    pltpu.sync_copy(iidx_hbm.at[pl.ds(base, _BPW)], iidx_v)

    rows_lo = lax.iota(jnp.int32, LANES)
    rows_hi = rows_lo + LANES

    def fire(uvec, ivec, l):
        s = l % DEPTH
        off_u = pl.multiple_of((uvec[l] >> 7) << 7, TILE)
        off_i = pl.multiple_of((ivec[l] >> 7) << 7, TILE)
        pltpu.async_copy(ut_hbm.at[:, pl.ds(off_u, TILE)], ubufs.at[s],
                         semus[s])
        pltpu.async_copy(it_hbm.at[:, pl.ds(off_i, TILE)], ibufs.at[s],
                         semis[s])

    def drain_and_use(uvec, ivec, l, b):
        s = l % DEPTH
        pltpu.make_async_copy(ut_hbm.at[:, pl.ds(0, TILE)], ubufs.at[s],
                              semus[s]).wait()
        pltpu.make_async_copy(it_hbm.at[:, pl.ds(0, TILE)], ibufs.at[s],
                              semis[s]).wait()
        cu_vec = jnp.full((LANES,), uvec[l] & 127, dtype=jnp.int32)
        ci_vec = jnp.full((LANES,), ivec[l] & 127, dtype=jnp.int32)
        b_vec = jnp.full((LANES,), b, dtype=jnp.int32)
        for rows in (rows_lo, rows_hi):
            uv = plsc.load_gather(ubufs.at[s], [rows, cu_vec])
            iv = plsc.load_gather(ibufs.at[s], [rows, ci_vec])
            plsc.store_scatter(ocols, [rows, b_vec], uv * iv)

    uvec0 = uidx_v[pl.ds(0, LANES)]
    ivec0 = iidx_v[pl.ds(0, LANES)]
    for l in range(DEPTH):
        fire(uvec0, ivec0, l)

    def group(g, carry):
        uvec = uidx_v[pl.ds(g * LANES, LANES)]
        ivec = iidx_v[pl.ds(g * LANES, LANES)]
        uvec_n = uidx_v[pl.ds(g * LANES + LANES, LANES)]
        ivec_n = iidx_v[pl.ds(g * LANES + LANES, LANES)]
        for l in range(LANES):
            drain_and_use(uvec, ivec, l, g * LANES + l)
            if l < LANES - DEPTH:
                fire(uvec, ivec, l + DEPTH)
            else:
                fire(uvec_n, ivec_n, l + DEPTH - LANES)
        return carry

    lax.fori_loop(0, _NG - 1, group, 0)

    uvec = uidx_v[pl.ds((_NG - 1) * LANES, LANES)]
    ivec = iidx_v[pl.ds((_NG - 1) * LANES, LANES)]
    for l in range(LANES):
        drain_and_use(uvec, ivec, l, (_NG - 1) * LANES + l)
        if l < LANES - DEPTH:
            fire(uvec, ivec, l + DEPTH)

    pltpu.sync_copy(ocols, out_hbm.at[:, pl.ds(base, _BPW)])


def kernel(userIdx, itemIdx, uEmbd, iEmbd):
    out_t = _mf_sc(userIdx, itemIdx, uEmbd.T, iEmbd.T)
    return out_t.T
